# baseline (speedup 1.0000x reference)
import functools

import jax
import jax.numpy as jnp
from jax import lax
from jax.experimental import pallas as pl
from jax.experimental.pallas import tpu as pltpu
from jax.experimental.pallas import tpu_sc as plsc

EMBED_DIM = 64
NUM_CORES = 2
NUM_SUBCORES = 16
NUM_WORKERS = NUM_CORES * NUM_SUBCORES
CHUNK = 128


@functools.lru_cache(maxsize=None)
def _build(batch_total: int):
  assert batch_total % (NUM_WORKERS * 2 * CHUNK) == 0
  b_per_w = batch_total // NUM_WORKERS
  n_pairs = b_per_w // (2 * CHUNK)
  mesh = plsc.VectorSubcoreMesh(core_axis_name="c", subcore_axis_name="s")

  @functools.partial(
      pl.kernel,
      mesh=mesh,
      out_type=jax.ShapeDtypeStruct((batch_total, EMBED_DIM), jnp.float32),
      scratch_types=[
          pltpu.VMEM((CHUNK,), jnp.int32),
          pltpu.VMEM((CHUNK,), jnp.int32),
          pltpu.VMEM((CHUNK, EMBED_DIM), jnp.float32),
          pltpu.VMEM((CHUNK, EMBED_DIM), jnp.float32),
          pltpu.SemaphoreType.DMA,
          pltpu.SemaphoreType.DMA,
      ],
      compiler_params=pltpu.CompilerParams(use_tc_tiling_on_sc=False),
  )
  def gather_kernel(table_hbm, idx_hbm, out_hbm, idx_v0, idx_v1, rows_v0,
                    rows_v1, sem0, sem1):
    wid = lax.axis_index("s") * NUM_CORES + lax.axis_index("c")
    base = wid * b_per_w
    slots = ((idx_v0, rows_v0, sem0), (idx_v1, rows_v1, sem1))

    def start(off, slot):
      idx_v, rows_v, sem = slots[slot]
      pltpu.sync_copy(idx_hbm.at[pl.ds(off, CHUNK)], idx_v)
      pltpu.async_copy(table_hbm.at[idx_v], rows_v, sem)

    def finish(off, slot):
      idx_v, rows_v, sem = slots[slot]
      pltpu.make_async_copy(table_hbm.at[idx_v], rows_v, sem).wait()
      pltpu.sync_copy(rows_v, out_hbm.at[pl.ds(off, CHUNK)])

    start(base, 0)

    def body(j, _):
      off = base + j * (2 * CHUNK)
      start(off + CHUNK, 1)
      finish(off, 0)

      @pl.when(j + 1 < n_pairs)
      def _():
        start(off + 2 * CHUNK, 0)

      finish(off + CHUNK, 1)
      return 0

    lax.fori_loop(0, n_pairs, body, 0)

  return gather_kernel


def kernel(inputs, embeddings):
  batch, fields = inputs.shape
  idx_flat = inputs.astype(jnp.int32).reshape(batch * fields)
  out_flat = _build(batch * fields)(embeddings, idx_flat)
  return out_flat.reshape(batch, fields, EMBED_DIM)


# speedup vs baseline: 1.0393x; 1.0393x over previous
import functools

import jax
import jax.numpy as jnp
from jax import lax
from jax.experimental import pallas as pl
from jax.experimental.pallas import tpu as pltpu
from jax.experimental.pallas import tpu_sc as plsc

EMBED_DIM = 64
NUM_CORES = 2
NUM_SUBCORES = 16
NUM_WORKERS = NUM_CORES * NUM_SUBCORES
CHUNK = 128
NBUF = 8


@functools.lru_cache(maxsize=None)
def _build(batch_total: int):
  assert batch_total % (NUM_WORKERS * NBUF * CHUNK) == 0
  b_per_w = batch_total // NUM_WORKERS
  n_rounds = b_per_w // (NBUF * CHUNK)
  mesh = plsc.VectorSubcoreMesh(core_axis_name="c", subcore_axis_name="s")

  scratch = (
      [pltpu.VMEM((CHUNK,), jnp.int32) for _ in range(NBUF)]
      + [pltpu.VMEM((CHUNK, EMBED_DIM), jnp.float32) for _ in range(NBUF)]
      + [pltpu.SemaphoreType.DMA for _ in range(3 * NBUF)]
  )

  @functools.partial(
      pl.kernel,
      mesh=mesh,
      out_type=jax.ShapeDtypeStruct((batch_total, EMBED_DIM), jnp.float32),
      scratch_types=scratch,
      compiler_params=pltpu.CompilerParams(use_tc_tiling_on_sc=False),
  )
  def gather_kernel(table_hbm, idx_hbm, out_hbm, *scr):
    stage = scr[:NBUF]
    rows = scr[NBUF:2 * NBUF]
    isem = scr[2 * NBUF:3 * NBUF]
    gsem = scr[3 * NBUF:4 * NBUF]
    wsem = scr[4 * NBUF:5 * NBUF]
    wid = lax.axis_index("s") * NUM_CORES + lax.axis_index("c")
    base = wid * b_per_w

    def idx_copy(i, s):
      return pltpu.make_async_copy(idx_hbm.at[pl.ds(base + i * CHUNK, CHUNK)],
                                   stage[s], isem[s])

    def gather(i, s):
      del i
      return pltpu.make_async_copy(table_hbm.at[stage[s]], rows[s], gsem[s])

    def write(i, s):
      return pltpu.make_async_copy(
          rows[s], out_hbm.at[pl.ds(base + i * CHUNK, CHUNK)], wsem[s])

    for s in range(NBUF):
      idx_copy(s, s).start()

    def body(r, _):
      i0 = r * NBUF
      for s in range(NBUF):
        idx_copy(i0 + s, s).wait()

        @pl.when(r > 0)
        def _():
          write(i0 + s - NBUF, s).wait()

        gather(i0 + s, s).start()
      for s in range(NBUF):
        gather(i0 + s, s).wait()
        write(i0 + s, s).start()

        @pl.when(r + 1 < n_rounds)
        def _():
          idx_copy(i0 + s + NBUF, s).start()

      return 0

    lax.fori_loop(0, n_rounds, body, 0)
    for s in range(NBUF):
      write((n_rounds - 1) * NBUF + s, s).wait()

  return gather_kernel


def kernel(inputs, embeddings):
  batch, fields = inputs.shape
  idx_flat = inputs.astype(jnp.int32).reshape(batch * fields)
  out_flat = _build(batch * fields)(embeddings, idx_flat)
  return out_flat.reshape(batch, fields, EMBED_DIM)
